# baseline (speedup 1.0000x reference)
import functools
import math

import jax
import jax.numpy as jnp
from jax import lax
from jax.experimental import pallas as pl
from jax.experimental.pallas import tpu as pltpu
from jax.experimental.pallas import tpu_sc as plsc


def _qnorm_body(x_ref, wq_ref, n_ref):
    q = jnp.dot(x_ref[...], wq_ref[...], preferred_element_type=jnp.float32)
    sumsq = jnp.sum(q * q, axis=1)
    n_ref[...] = jnp.sqrt(sumsq).reshape(n_ref.shape)


def _q_norms(x_flat, Wq, block_rows=1024):
    rows, embed = x_flat.shape
    head = Wq.shape[1]
    grid = (rows // block_rows,)
    norms = pl.pallas_call(
        _qnorm_body,
        grid=grid,
        in_specs=[
            pl.BlockSpec((block_rows, embed), lambda i: (i, 0)),
            pl.BlockSpec((embed, head), lambda i: (0, 0)),
        ],
        out_specs=pl.BlockSpec((1, 1, block_rows), lambda i: (i, 0, 0)),
        out_shape=jax.ShapeDtypeStruct((rows // block_rows, 1, block_rows),
                                       jnp.float32),
    )(x_flat, Wq)
    return norms.reshape(rows)


def _sc_gather(x_flat, gidx):
    info = plsc.get_sparse_core_info()
    num_workers = info.num_cores * info.num_subcores
    n_idx = gidx.shape[0]
    rpw = n_idx // num_workers
    embed = x_flat.shape[1]
    mesh = plsc.VectorSubcoreMesh(core_axis_name="c", subcore_axis_name="s")

    @functools.partial(
        pl.kernel,
        mesh=mesh,
        out_type=jax.ShapeDtypeStruct((n_idx, embed), jnp.float32),
    scratch_types=[
            pltpu.VMEM((rpw,), jnp.int32),
            pltpu.VMEM((rpw, embed), jnp.float32),
            pltpu.SemaphoreType.DMA,
            pltpu.SemaphoreType.DMA,
        ],
    )
    def gather_kernel(x_hbm, idx_hbm, xt_hbm, idx_v, xrow_v, sem_a, sem_b):
        wid = lax.axis_index("s") * info.num_cores + lax.axis_index("c")
        base = wid * rpw
        pltpu.sync_copy(idx_hbm.at[pl.ds(base, rpw)], idx_v)
        half = rpw // 2
        iv0 = idx_v[pl.ds(0, half)]
        iv1 = idx_v[pl.ds(half, half)]
        g0 = pltpu.async_copy(x_hbm.at[iv0], xrow_v.at[pl.ds(0, half)], sem_a)
        g1 = pltpu.async_copy(x_hbm.at[iv1], xrow_v.at[pl.ds(half, half)],
                              sem_b)
        g0.wait()
        w0 = pltpu.async_copy(xrow_v.at[pl.ds(0, half)],
                              xt_hbm.at[pl.ds(base, half)], sem_a)
        g1.wait()
        w1 = pltpu.async_copy(xrow_v.at[pl.ds(half, half)],
                              xt_hbm.at[pl.ds(base + half, half)], sem_b)
        w0.wait()
        w1.wait()

    return gather_kernel(x_flat, gidx)


def _attn_body(topk, xt_ref, wq_ref, wk_ref, wv_ref, o_ref):
    x = xt_ref[...]
    q = jnp.dot(x, wq_ref[...], preferred_element_type=jnp.float32)
    k = jnp.dot(x, wk_ref[...], preferred_element_type=jnp.float32)
    v = jnp.dot(x, wv_ref[...], preferred_element_type=jnp.float32)
    head = q.shape[1]
    w = lax.dot_general(q, k, (((1,), (1,)), ((), ())),
                        preferred_element_type=jnp.float32)
    w = w * (1.0 / math.sqrt(head))
    n = w.shape[0]
    col = lax.broadcasted_iota(jnp.int32, (n, n), 1)
    w = jnp.where(col < topk, w, -jnp.inf)
    w = w - jnp.max(w, axis=-1, keepdims=True)
    e = jnp.exp(w)
    p = e / jnp.sum(e, axis=-1, keepdims=True)
    o_ref[...] = jnp.dot(p, v, preferred_element_type=jnp.float32)


def _attention(x_top, Wq, Wk, Wv, batches, rows_per_batch, topk):
    embed = x_top.shape[1]
    head = Wq.shape[1]
    wspec = pl.BlockSpec((embed, head), lambda b: (0, 0))
    return pl.pallas_call(
        functools.partial(_attn_body, topk),
        grid=(batches,),
        in_specs=[
            pl.BlockSpec((rows_per_batch, embed), lambda b: (b, 0)),
            wspec, wspec, wspec,
        ],
        out_specs=pl.BlockSpec((rows_per_batch, head), lambda b: (b, 0)),
        out_shape=jax.ShapeDtypeStruct((batches * rows_per_batch, head),
                                       jnp.float32),
    )(x_top, Wq, Wk, Wv)


def _sc_scatter(rows, sidx, region_rows):
    info = plsc.get_sparse_core_info()
    n_cores, n_sub = info.num_cores, info.num_subcores
    n_idx = sidx.shape[0]
    rpw = n_idx // (n_cores * n_sub)
    zpw = region_rows // n_sub
    head = rows.shape[1]
    mesh = plsc.VectorSubcoreMesh(core_axis_name="c", subcore_axis_name="s")
    zeros_blk = jnp.zeros((zpw, head), jnp.float32)

    @functools.partial(
        pl.kernel,
        mesh=mesh,
        out_type=jax.ShapeDtypeStruct((n_cores * region_rows, head),
                                      jnp.float32),
        scratch_types=[
            pltpu.VMEM((rpw,), jnp.int32),
            pltpu.VMEM((rpw, head), jnp.float32),
            pltpu.VMEM((zpw, head), jnp.float32),
            pltpu.SemaphoreType.DMA,
            pltpu.SemaphoreType.DMA,
            pltpu.SemaphoreType.DMA,
        ],
    )
    def scatter_kernel(rows_hbm, idx_hbm, zeros_hbm, out_hbm,
                       idx_v, row_v, zero_v, sem_z, sem_i, sem_r):
        cid = lax.axis_index("c")
        sid = lax.axis_index("s")
        cz = pltpu.async_copy(zeros_hbm, zero_v, sem_z)
        base = (cid * n_sub + sid) * rpw
        ci = pltpu.async_copy(idx_hbm.at[pl.ds(base, rpw)], idx_v, sem_i)
        cr = pltpu.async_copy(rows_hbm.at[pl.ds(base, rpw)], row_v, sem_r)
        cz.wait()
        zbase = cid * region_rows + sid * zpw
        pltpu.sync_copy(zero_v, out_hbm.at[pl.ds(zbase, zpw)])
        plsc.subcore_barrier()
        ci.wait()
        cr.wait()
        pltpu.async_copy(row_v, out_hbm.at[idx_v], sem_z).wait()

    return scatter_kernel(rows, sidx, zeros_blk)


def kernel(index, Wq, Wk, Wv):
    B, T, E = index.shape
    head = Wq.shape[1]
    topk = int(0.1 * T)
    topk_pad = 512

    x_flat = index.reshape(B * T, E)
    norms = _q_norms(x_flat, Wq)
    _, top_idx = lax.top_k(norms.reshape(B, T), topk)
    top_idx = top_idx.astype(jnp.int32)

    flat_idx = top_idx + (jnp.arange(B, dtype=jnp.int32) * T)[:, None]
    pad = jnp.broadcast_to(flat_idx[:, :1], (B, topk_pad - topk))
    full_idx = jnp.concatenate([flat_idx, pad], axis=1).reshape(-1)

    x_top = _sc_gather(x_flat, full_idx)
    out_top = _attention(x_top, Wq, Wk, Wv, B, topk_pad, topk)
    out_flat = _sc_scatter(out_top, full_idx, T)
    return out_flat.reshape(B, T, head)


# speedup vs baseline: 1.0081x; 1.0081x over previous
import functools
import math

import jax
import jax.numpy as jnp
from jax import lax
from jax.experimental import pallas as pl
from jax.experimental.pallas import tpu as pltpu
from jax.experimental.pallas import tpu_sc as plsc


def _qnorm_body(x_ref, wq_ref, n_ref):
    q = jnp.dot(x_ref[...], wq_ref[...], preferred_element_type=jnp.float32)
    sumsq = jnp.sum(q * q, axis=1)
    n_ref[...] = jnp.broadcast_to(jnp.sqrt(sumsq)[:, None], n_ref.shape)


def _q_norms(x_flat, Wq, block_rows=1024):
    rows, embed = x_flat.shape
    head = Wq.shape[1]
    grid = (rows // block_rows,)
    norms = pl.pallas_call(
        _qnorm_body,
        grid=grid,
        in_specs=[
            pl.BlockSpec((block_rows, embed), lambda i: (i, 0)),
            pl.BlockSpec((embed, head), lambda i: (0, 0)),
        ],
        out_specs=pl.BlockSpec((block_rows, 128), lambda i: (i, 0)),
        out_shape=jax.ShapeDtypeStruct((rows, 128), jnp.float32),
    )(x_flat, Wq)
    return norms[:, 0]


def _sc_gather(x_flat, gidx):
    info = plsc.get_sparse_core_info()
    num_workers = info.num_cores * info.num_subcores
    n_idx = gidx.shape[0]
    rpw = n_idx // num_workers
    embed = x_flat.shape[1]
    mesh = plsc.VectorSubcoreMesh(core_axis_name="c", subcore_axis_name="s")

    @functools.partial(
        pl.kernel,
        mesh=mesh,
        out_type=jax.ShapeDtypeStruct((n_idx, embed), jnp.float32),
    scratch_types=[
            pltpu.VMEM((rpw,), jnp.int32),
            pltpu.VMEM((rpw, embed), jnp.float32),
            pltpu.SemaphoreType.DMA,
            pltpu.SemaphoreType.DMA,
        ],
    )
    def gather_kernel(x_hbm, idx_hbm, xt_hbm, idx_v, xrow_v, sem_a, sem_b):
        wid = lax.axis_index("s") * info.num_cores + lax.axis_index("c")
        base = wid * rpw
        pltpu.sync_copy(idx_hbm.at[pl.ds(base, rpw)], idx_v)
        half = rpw // 2
        iv0 = idx_v[pl.ds(0, half)]
        iv1 = idx_v[pl.ds(half, half)]
        g0 = pltpu.async_copy(x_hbm.at[iv0], xrow_v.at[pl.ds(0, half)], sem_a)
        g1 = pltpu.async_copy(x_hbm.at[iv1], xrow_v.at[pl.ds(half, half)],
                              sem_b)
        g0.wait()
        w0 = pltpu.async_copy(xrow_v.at[pl.ds(0, half)],
                              xt_hbm.at[pl.ds(base, half)], sem_a)
        g1.wait()
        w1 = pltpu.async_copy(xrow_v.at[pl.ds(half, half)],
                              xt_hbm.at[pl.ds(base + half, half)], sem_b)
        w0.wait()
        w1.wait()

    return gather_kernel(x_flat, gidx)


def _attn_body(topk, xt_ref, wq_ref, wk_ref, wv_ref, o_ref):
    x = xt_ref[...]
    q = jnp.dot(x, wq_ref[...], preferred_element_type=jnp.float32)
    k = jnp.dot(x, wk_ref[...], preferred_element_type=jnp.float32)
    v = jnp.dot(x, wv_ref[...], preferred_element_type=jnp.float32)
    head = q.shape[1]
    w = lax.dot_general(q, k, (((1,), (1,)), ((), ())),
                        preferred_element_type=jnp.float32)
    w = w * (1.0 / math.sqrt(head))
    n = w.shape[0]
    col = lax.broadcasted_iota(jnp.int32, (n, n), 1)
    w = jnp.where(col < topk, w, -jnp.inf)
    w = w - jnp.max(w, axis=-1, keepdims=True)
    e = jnp.exp(w)
    p = e / jnp.sum(e, axis=-1, keepdims=True)
    o_ref[...] = jnp.dot(p, v, preferred_element_type=jnp.float32)


def _attention(x_top, Wq, Wk, Wv, batches, rows_per_batch, topk):
    embed = x_top.shape[1]
    head = Wq.shape[1]
    wspec = pl.BlockSpec((embed, head), lambda b: (0, 0))
    return pl.pallas_call(
        functools.partial(_attn_body, topk),
        grid=(batches,),
        in_specs=[
            pl.BlockSpec((rows_per_batch, embed), lambda b: (b, 0)),
            wspec, wspec, wspec,
        ],
        out_specs=pl.BlockSpec((rows_per_batch, head), lambda b: (b, 0)),
        out_shape=jax.ShapeDtypeStruct((batches * rows_per_batch, head),
                                       jnp.float32),
    )(x_top, Wq, Wk, Wv)


def _sc_scatter(rows, sidx, region_rows):
    info = plsc.get_sparse_core_info()
    n_cores, n_sub = info.num_cores, info.num_subcores
    n_idx = sidx.shape[0]
    rpw = n_idx // (n_cores * n_sub)
    zpw = region_rows // n_sub
    head = rows.shape[1]
    mesh = plsc.VectorSubcoreMesh(core_axis_name="c", subcore_axis_name="s")
    zeros_blk = jnp.zeros((zpw, head), jnp.float32)

    @functools.partial(
        pl.kernel,
        mesh=mesh,
        out_type=jax.ShapeDtypeStruct((n_cores * region_rows, head),
                                      jnp.float32),
        scratch_types=[
            pltpu.VMEM((rpw,), jnp.int32),
            pltpu.VMEM((rpw, head), jnp.float32),
            pltpu.VMEM((zpw, head), jnp.float32),
            pltpu.SemaphoreType.DMA,
            pltpu.SemaphoreType.DMA,
            pltpu.SemaphoreType.DMA,
        ],
    )
    def scatter_kernel(rows_hbm, idx_hbm, zeros_hbm, out_hbm,
                       idx_v, row_v, zero_v, sem_z, sem_i, sem_r):
        cid = lax.axis_index("c")
        sid = lax.axis_index("s")
        cz = pltpu.async_copy(zeros_hbm, zero_v, sem_z)
        base = (cid * n_sub + sid) * rpw
        ci = pltpu.async_copy(idx_hbm.at[pl.ds(base, rpw)], idx_v, sem_i)
        cr = pltpu.async_copy(rows_hbm.at[pl.ds(base, rpw)], row_v, sem_r)
        cz.wait()
        zbase = cid * region_rows + sid * zpw
        pltpu.sync_copy(zero_v, out_hbm.at[pl.ds(zbase, zpw)])
        plsc.subcore_barrier()
        ci.wait()
        cr.wait()
        pltpu.async_copy(row_v, out_hbm.at[idx_v], sem_z).wait()

    return scatter_kernel(rows, sidx, zeros_blk)


def kernel(index, Wq, Wk, Wv):
    B, T, E = index.shape
    head = Wq.shape[1]
    topk = int(0.1 * T)
    topk_pad = 512

    x_flat = index.reshape(B * T, E)
    norms = _q_norms(x_flat, Wq)
    _, top_idx = lax.top_k(norms.reshape(B, T), topk)
    top_idx = top_idx.astype(jnp.int32)

    flat_idx = top_idx + (jnp.arange(B, dtype=jnp.int32) * T)[:, None]
    pad = jnp.broadcast_to(flat_idx[:, :1], (B, topk_pad - topk))
    full_idx = jnp.concatenate([flat_idx, pad], axis=1).reshape(-1)

    x_top = _sc_gather(x_flat, full_idx)
    out_top = _attention(x_top, Wq, Wk, Wv, B, topk_pad, topk)
    out_flat = _sc_scatter(out_top, full_idx, T)
    return out_flat.reshape(B, T, head)
